# baseline (speedup 1.0000x reference)
import jax
import jax.numpy as jnp
from jax import lax
from jax.experimental import pallas as pl
from jax.experimental.pallas import tpu as pltpu
from jax.experimental.pallas import tpu_sc as plsc

VOCAB = 30522
HID = 768
PROMPT = 20
B = 32
S = 512
EPS = 1e-12
L = 16
NH = HID // L
NW = 32
SW = S // NW


def _lane_perm(x, idx):
    dn = lax.GatherDimensionNumbers(offset_dims=(), collapsed_slice_dims=(0,),
                                    start_index_map=(0,))
    return lax.gather(x, idx[:, None], dn, slice_sizes=(1,),
                      mode=lax.GatherScatterMode.PROMISE_IN_BOUNDS)


def _allsum(v):
    idx = lax.iota(jnp.int32, L)
    for sft in (8, 4, 2, 1):
        v = v + _lane_perm(v, jnp.bitwise_xor(idx, sft))
    return v


def _rsqrt_vec(v):
    i = lax.bitcast_convert_type(v, jnp.int32)
    i = jnp.int32(0x5F3759DF) - lax.shift_right_arithmetic(
        i, jnp.full((L,), 1, jnp.int32))
    y = lax.bitcast_convert_type(i, jnp.float32)
    for _ in range(3):
        y = y * (1.5 - 0.5 * v * y * y)
    return y


def _sc_body(ids_hbm, word_hbm, pos_hbm, type_hbm, prompt_hbm, gamma_hbm,
             beta_hbm, out_hbm, idx_v, rows_v, pos_v, prompt_v, type_v,
             gamma_v, beta_v, sem):
    cid = lax.axis_index("c")
    sid = lax.axis_index("s")
    wid = sid * 2 + cid
    s0 = wid * SW

    pltpu.sync_copy(ids_hbm.at[pl.ds(wid * (B * SW), B * SW)], idx_v)
    pltpu.sync_copy(pos_hbm.at[pl.ds(s0, SW)], pos_v)
    pltpu.sync_copy(type_hbm.at[0], type_v)
    pltpu.sync_copy(gamma_hbm, gamma_v)
    pltpu.sync_copy(beta_hbm, beta_v)

    has_prompt = wid <= 1

    @pl.when(has_prompt)
    def _():
        pltpu.sync_copy(prompt_hbm.at[jnp.minimum(wid, 1)], prompt_v)

    def _addtype(t, c):
        for j in range(NH):
            sl = pl.ds(j * L, L)
            pos_v[t, sl] = pos_v[t, sl] + type_v[sl]
        return c
    lax.fori_loop(0, SW, _addtype, 0)

    def _make_token(use_prompt):
        def _token(t, c2):
            if use_prompt:
                s = s0 + t
                inp = jnp.logical_and(s >= 1, s < 1 + PROMPT)
                pm = jnp.full((L,), jnp.where(inp, 1.0, 0.0), jnp.float32)
            acc = jnp.zeros((L,), jnp.float32)
            acc2 = jnp.zeros((L,), jnp.float32)
            for j in range(NH):
                sl = pl.ds(j * L, L)
                xr = rows_v[t, sl]
                if use_prompt:
                    xr = xr + pm * (prompt_v[t, sl] - xr)
                x = xr + pos_v[t, sl]
                rows_v[t, sl] = x
                acc = acc + x
                acc2 = acc2 + x * x
            meanv = _allsum(acc) * (1.0 / HID)
            varv = _allsum(acc2) * (1.0 / HID) - meanv * meanv
            rstd = _rsqrt_vec(varv + EPS)
            for j in range(NH):
                sl = pl.ds(j * L, L)
                rows_v[t, sl] = ((rows_v[t, sl] - meanv) * rstd
                                 * gamma_v[sl] + beta_v[sl])
            return c2
        return _token

    def _batch(b, c):
        pltpu.async_copy(word_hbm.at[idx_v.at[pl.ds(b * SW, SW)]],
                         rows_v, sem).wait()

        @pl.when(has_prompt)
        def _():
            lax.fori_loop(0, SW, _make_token(True), 0)

        @pl.when(jnp.logical_not(has_prompt))
        def _():
            lax.fori_loop(0, SW, _make_token(False), 0)

        pltpu.sync_copy(rows_v, out_hbm.at[b, pl.ds(s0, SW)])
        return c
    lax.fori_loop(0, B, _batch, 0)


def kernel(input_ids, word_emb, pos_emb, type_emb, prompt_emb, gamma, beta):
    ids_flat = (input_ids.reshape(B, NW, SW).transpose(1, 0, 2)
                .reshape(NW * B * SW))
    t = jnp.arange(SW)
    prompt_tiles = jnp.stack([
        prompt_emb[jnp.clip(t - 1, 0, PROMPT - 1)],
        prompt_emb[jnp.clip(t + SW - 1, 0, PROMPT - 1)],
    ])

    mesh = plsc.VectorSubcoreMesh(core_axis_name="c", subcore_axis_name="s")
    f = pl.kernel(
        _sc_body,
        out_type=jax.ShapeDtypeStruct((B, S, HID), jnp.float32),
        mesh=mesh,
        scratch_types=[
            pltpu.VMEM((B * SW,), jnp.int32),
            pltpu.VMEM((SW, HID), jnp.float32),
            pltpu.VMEM((SW, HID), jnp.float32),
            pltpu.VMEM((SW, HID), jnp.float32),
            pltpu.VMEM((HID,), jnp.float32),
            pltpu.VMEM((HID,), jnp.float32),
            pltpu.VMEM((HID,), jnp.float32),
            pltpu.SemaphoreType.DMA,
        ],
    )
    return f(ids_flat, word_emb, pos_emb, type_emb, prompt_tiles, gamma, beta)
